# Initial kernel scaffold; baseline (speedup 1.0000x reference)
import jax
import jax.numpy as jnp
from jax.experimental import pallas as pl


def kernel(batch, Wn, bn, Wf, bf, t0, W1_0, b1_0, g0, be0, W2_0, b2_0, t1, W1_1, b1_1, g1, be1, W2_1, b2_1):
    raise NotImplementedError("write your pallas kernel here")


# speedup vs baseline: 823.7743x; 823.7743x over previous
import functools

import jax
import jax.numpy as jnp
from jax.experimental import pallas as pl

_B, _N, _F_IN, _H, _OUT = 4, 512, 128, 64, 64


def _dot(a, b):
    return jax.lax.dot_general(
        a, b, (((1,), (0,)), ((), ())), preferred_element_type=jnp.float32
    )


def _fwd_kernel(
    x_ref, Wn_ref, bn_ref, Wf_ref, bf_ref,
    t0_ref, W10_ref, b10_ref, g0_ref, be0_ref, W20_ref, b20_ref,
    t1_ref, W11_ref, b11_ref, g1_ref, be1_ref, W21_ref, b21_ref,
    out_ref,
):
    ntot = _B * _N
    x = _dot(x_ref[:], Wn_ref[:]) + bn_ref[:]

    row = jax.lax.broadcasted_iota(jnp.int32, (ntot, 1), 0)
    in_graph = row < _N

    layers = (
        (t0_ref, W10_ref, b10_ref, g0_ref, be0_ref, W20_ref, b20_ref),
        (t1_ref, W11_ref, b11_ref, g1_ref, be1_ref, W21_ref, b21_ref),
    )
    for (t_ref, W1_ref, b1_ref, g_ref, be_ref, W2_ref, b2_ref) in layers:
        h = jnp.maximum(x, 0.0)
        msg = h[: _N, :] + 1e-7
        gate = msg * t_ref[0, 0]
        m = jnp.max(gate, axis=0, keepdims=True)
        e = jnp.exp(gate - m)
        denom = jnp.sum(e, axis=0, keepdims=True)
        aggr = jnp.sum(msg * e, axis=0, keepdims=True) / (denom + 1e-16)
        out = h + jnp.where(in_graph, aggr, 0.0)
        hh = _dot(out, W1_ref[:]) + b1_ref[:]
        mu = jnp.mean(hh, axis=-1, keepdims=True)
        var = jnp.mean((hh - mu) ** 2, axis=-1, keepdims=True)
        hh = (hh - mu) / jnp.sqrt(var + 1e-5) * g_ref[:] + be_ref[:]
        hh = jnp.maximum(hh, 0.0)
        x = x + _dot(hh, W2_ref[:]) + b2_ref[:]
    y = jnp.maximum(x, 0.0)
    out_ref[:] = _dot(y, Wf_ref[:]) + bf_ref[:]


@functools.partial(jax.jit, static_argnames=())
def kernel(batch, Wn, bn, Wf, bf, t0, W1_0, b1_0, g0, be0, W2_0, b2_0,
           t1, W1_1, b1_1, g1, be1, W2_1, b2_1):
    b, n, f = batch.shape
    x = batch.reshape(b * n, f)
    r2 = lambda v: v.reshape(1, -1)
    out = pl.pallas_call(
        _fwd_kernel,
        out_shape=jax.ShapeDtypeStruct((b * n, _OUT), jnp.float32),
    )(
        x, Wn, r2(bn), Wf, r2(bf),
        t0.reshape(1, 1), W1_0, r2(b1_0), r2(g0), r2(be0), W2_0, r2(b2_0),
        t1.reshape(1, 1), W1_1, r2(b1_1), r2(g1), r2(be1), W2_1, r2(b2_1),
    )
    return out.reshape(b, n, _OUT)
